# baseline (speedup 1.0000x reference)
import functools

import jax
import jax.numpy as jnp
from jax import lax
from jax.experimental import pallas as pl
from jax.experimental.pallas import tpu as pltpu
from jax.experimental.pallas import tpu_sc as plsc

VOC = 1_000_000
EMB = 64
B = 16384
K = 20
R = 20
KR = K + R
PAIR = 2 * EMB

NC = 2
NS = 16
NW = NC * NS
NLANE = 16
NV = EMB // NLANE

NB = B // NW
SB = 8
NSUB = NB // SB
TASKS = SB * KR
GCHUNK = 64
NG = TASKS // GCHUNK
NGRP = TASKS // NLANE

TBLK = 512
NTB = (VOC + TBLK - 1) // TBLK
NPROW = NTB * (TBLK // 2)


def _tc_tpose_body(x_ref, o_ref):
    x = x_ref[...]
    o_ref[...] = jnp.concatenate(
        [x[:, :TBLK // 2].T, x[:, TBLK // 2:].T], axis=1)


_tc_tpose = pl.pallas_call(
    _tc_tpose_body,
    grid=(NTB,),
    in_specs=[pl.BlockSpec((EMB, TBLK), lambda i: (0, i))],
    out_specs=pl.BlockSpec((TBLK // 2, PAIR), lambda i: (i, 0)),
    out_shape=jax.ShapeDtypeStruct((NPROW, PAIR), jnp.float32),
)


@functools.partial(
    pl.kernel,
    out_type=jax.ShapeDtypeStruct((B * KR,), jnp.float32),
    mesh=plsc.VectorSubcoreMesh(core_axis_name="c", subcore_axis_name="s"),
    compiler_params=pltpu.CompilerParams(
        needs_layout_passes=False, use_tc_tiling_on_sc=True),
    scratch_types=[
        pltpu.VMEM((NB,), jnp.int32),
        pltpu.VMEM((NB * KR,), jnp.int32),
        pltpu.VMEM((NLANE,), jnp.int32),
        pltpu.VMEM((NLANE,), jnp.int32),
        pltpu.VMEM((TASKS,), jnp.int32),
        pltpu.VMEM((TASKS,), jnp.int32),
        pltpu.VMEM((SB, PAIR), jnp.float32),
        pltpu.VMEM((SB, PAIR), jnp.float32),
        pltpu.VMEM((TASKS, PAIR), jnp.float32),
        pltpu.VMEM((TASKS, PAIR), jnp.float32),
        pltpu.VMEM((TASKS * NLANE,), jnp.float32),
        pltpu.VMEM((TASKS,), jnp.float32),
        pltpu.SemaphoreType.DMA,
        pltpu.SemaphoreType.DMA,
    ],
)
def _sc_dots(center_hbm, cw_hbm, emb_hbm, lw_hbm, dots_hbm,
             cidx, widx, cp0, cp1, wp0, wp1, eb0, eb1, wb0, wb1,
             pbuf, dbuf, sem0, sem1):
    wid = lax.axis_index("s") * NC + lax.axis_index("c")
    b0 = pl.multiple_of(wid * NB, NB)
    t0 = pl.multiple_of(wid * (NB * KR), NB * KR)
    pltpu.sync_copy(center_hbm.at[pl.ds(b0, NB)], cidx)
    pltpu.sync_copy(cw_hbm.at[pl.ds(t0, NB * KR)], widx)

    lane = lax.iota(jnp.int32, NLANE)
    zero16 = jnp.zeros((NLANE,), jnp.int32)

    def _pair_id(v):
        return (lax.shift_right_logical(v, 9) * 256) + (v & 255)

    def _stage_idx(s, cp, wp):
        sb0 = pl.multiple_of(s * SB, SB)
        st0 = pl.multiple_of(s * TASKS, TASKS)
        cp[:] = _pair_id(
            plsc.load_gather(cidx, [jnp.minimum(sb0 + lane, NB - 1)]))

        @pl.loop(0, NGRP)
        def _i(i):
            o = pl.multiple_of(i * NLANE, NLANE)
            wp[pl.ds(o, NLANE)] = _pair_id(widx[pl.ds(st0 + o, NLANE)])

    def _copies(cp, wp, eb, wb, sem):
        yield pltpu.make_async_copy(emb_hbm.at[cp.at[pl.ds(0, SB)]], eb, sem)
        for q in range(NG):
            yield pltpu.make_async_copy(
                lw_hbm.at[wp.at[pl.ds(q * GCHUNK, GCHUNK)]],
                wb.at[pl.ds(q * GCHUNK, GCHUNK)], sem)

    def _issue(s, cp, wp, eb, wb, sem):
        _stage_idx(s, cp, wp)
        for c in _copies(cp, wp, eb, wb, sem):
            c.start()

    def _wait(cp, wp, eb, wb, sem):
        for c in _copies(cp, wp, eb, wb, sem):
            c.wait()

    def _compute(s, eb, wb):
        sb0 = pl.multiple_of(s * SB, SB)
        st0 = pl.multiple_of(s * TASKS, TASKS)

        @pl.loop(0, SB)
        def _per_b(b):
            ch = lax.shift_right_logical(
                plsc.load_gather(cidx, [zero16 + (sb0 + b)]), 8) & 1
            codd = ch == 1
            e = [jnp.where(codd,
                           eb[b, pl.ds(EMB + j * NLANE, NLANE)],
                           eb[b, pl.ds(j * NLANE, NLANE)])
                 for j in range(NV)]

            @pl.loop(0, KR)
            def _per_k(k):
                t = b * KR + k
                wh = lax.shift_right_logical(
                    plsc.load_gather(widx, [zero16 + (st0 + t)]), 8) & 1
                wodd = wh == 1
                p = jnp.where(wodd,
                              wb[t, pl.ds(EMB, NLANE)],
                              wb[t, pl.ds(0, NLANE)]) * e[0]
                for j in range(1, NV):
                    p = p + jnp.where(
                        wodd,
                        wb[t, pl.ds(EMB + j * NLANE, NLANE)],
                        wb[t, pl.ds(j * NLANE, NLANE)]) * e[j]
                pbuf[pl.ds(pl.multiple_of(t * NLANE, NLANE), NLANE)] = p

        @pl.loop(0, NGRP)
        def _per_g(g):
            base = g * (NLANE * NLANE) + lane * NLANE
            acc = plsc.load_gather(pbuf, [base])
            for j in range(1, NLANE):
                acc = acc + plsc.load_gather(pbuf, [base + j])
            dbuf[pl.ds(pl.multiple_of(g * NLANE, NLANE), NLANE)] = acc

        pltpu.sync_copy(dbuf, dots_hbm.at[pl.ds(t0 + st0, TASKS)])

    _issue(0, cp0, wp0, eb0, wb0, sem0)

    @pl.loop(0, NSUB // 2)
    def _pair(h):
        s0 = h * 2
        _issue(s0 + 1, cp1, wp1, eb1, wb1, sem1)
        _wait(cp0, wp0, eb0, wb0, sem0)
        _compute(s0, eb0, wb0)

        @pl.when(h < NSUB // 2 - 1)
        def _():
            _issue(s0 + 2, cp0, wp0, eb0, wb0, sem0)

        _wait(cp1, wp1, eb1, wb1, sem1)
        _compute(s0 + 1, eb1, wb1)


def _tc_loss_body(d_ref, o_ref):
    d = d_ref[...]
    col = lax.broadcasted_iota(jnp.int32, (B, KR), 1)
    act = jax.nn.sigmoid(d)
    pos = -jnp.log(act)
    neg = -jnp.log(1.0 - act + 1e-3)
    is_pos = col < K
    s_pos = jnp.sum(jnp.where(is_pos, pos, 0.0))
    s_neg = jnp.sum(jnp.where(is_pos, 0.0, neg))
    o_ref[0, 0] = s_pos / (B * K) + s_neg / (B * R)


_tc_loss = pl.pallas_call(
    _tc_loss_body,
    out_shape=jax.ShapeDtypeStruct((1, 1), jnp.float32),
    out_specs=pl.BlockSpec(memory_space=pltpu.SMEM),
)


def kernel(center, context, rand, embeddings, linear_w):
    center = center.astype(jnp.int32)
    cw = jnp.concatenate([context, rand], axis=1).astype(jnp.int32)
    embR = _tc_tpose(embeddings.T)
    lwR = _tc_tpose(linear_w.T)
    dots = _sc_dots(center, cw.reshape(-1), embR, lwR)
    loss = _tc_loss(dots.reshape(B, KR))
    return loss[0, 0]


# speedup vs baseline: 3.1968x; 3.1968x over previous
import functools

import jax
import jax.numpy as jnp
from jax import lax
from jax.experimental import pallas as pl
from jax.experimental.pallas import tpu as pltpu
from jax.experimental.pallas import tpu_sc as plsc

VOC = 1_000_000
EMB = 64
B = 16384
K = 20
R = 20
KR = K + R
PAIR = 2 * EMB

NC = 2
NS = 16
NW = NC * NS
NLANE = 16
NV = EMB // NLANE

NB = B // NW
SB = 8
NSUB = NB // SB
TASKS = SB * KR
GCHUNK = 64
NG = TASKS // GCHUNK
NGRP = TASKS // NLANE

TBLK = 8192
NTB = (VOC + TBLK - 1) // TBLK
HBLK = TBLK // 2
NPROW = NTB * HBLK
SH_BLK = TBLK.bit_length() - 1
SH_HALF = HBLK.bit_length() - 1


def _tc_tpose_body(x_ref, o_ref):
    x = x_ref[...]
    o_ref[...] = jnp.concatenate([x[:, :HBLK].T, x[:, HBLK:].T], axis=1)


_tc_tpose = pl.pallas_call(
    _tc_tpose_body,
    grid=(NTB,),
    in_specs=[pl.BlockSpec((EMB, TBLK), lambda i: (0, i))],
    out_specs=pl.BlockSpec((HBLK, PAIR), lambda i: (i, 0)),
    out_shape=jax.ShapeDtypeStruct((NPROW, PAIR), jnp.float32),
)


@functools.partial(
    pl.kernel,
    out_type=jax.ShapeDtypeStruct((B * KR,), jnp.float32),
    mesh=plsc.VectorSubcoreMesh(core_axis_name="c", subcore_axis_name="s"),
    compiler_params=pltpu.CompilerParams(
        needs_layout_passes=False, use_tc_tiling_on_sc=True),
    scratch_types=[
        pltpu.VMEM((NB,), jnp.int32),
        pltpu.VMEM((NB * KR,), jnp.int32),
        pltpu.VMEM((NLANE,), jnp.int32),
        pltpu.VMEM((NLANE,), jnp.int32),
        pltpu.VMEM((TASKS,), jnp.int32),
        pltpu.VMEM((TASKS,), jnp.int32),
        pltpu.VMEM((SB, PAIR), jnp.float32),
        pltpu.VMEM((SB, PAIR), jnp.float32),
        pltpu.VMEM((TASKS, PAIR), jnp.float32),
        pltpu.VMEM((TASKS, PAIR), jnp.float32),
        pltpu.VMEM((TASKS * NLANE,), jnp.float32),
        pltpu.VMEM((TASKS,), jnp.float32),
        pltpu.SemaphoreType.DMA,
        pltpu.SemaphoreType.DMA,
    ],
)
def _sc_dots(center_hbm, cw_hbm, emb_hbm, lw_hbm, dots_hbm,
             cidx, widx, cp0, cp1, wp0, wp1, eb0, eb1, wb0, wb1,
             pbuf, dbuf, sem0, sem1):
    wid = lax.axis_index("s") * NC + lax.axis_index("c")
    b0 = pl.multiple_of(wid * NB, NB)
    t0 = pl.multiple_of(wid * (NB * KR), NB * KR)
    pltpu.sync_copy(center_hbm.at[pl.ds(b0, NB)], cidx)
    pltpu.sync_copy(cw_hbm.at[pl.ds(t0, NB * KR)], widx)

    lane = lax.iota(jnp.int32, NLANE)
    zero16 = jnp.zeros((NLANE,), jnp.int32)

    def _pair_id(v):
        return (lax.shift_right_logical(v, SH_BLK) * HBLK) + (v & (HBLK - 1))

    def _stage_idx(s, cp, wp):
        sb0 = pl.multiple_of(s * SB, SB)
        st0 = pl.multiple_of(s * TASKS, TASKS)
        cp[:] = _pair_id(
            plsc.load_gather(cidx, [jnp.minimum(sb0 + lane, NB - 1)]))

        @pl.loop(0, NGRP)
        def _i(i):
            o = pl.multiple_of(i * NLANE, NLANE)
            wp[pl.ds(o, NLANE)] = _pair_id(widx[pl.ds(st0 + o, NLANE)])

    def _copies(cp, wp, eb, wb, sem):
        yield pltpu.make_async_copy(emb_hbm.at[cp.at[pl.ds(0, SB)]], eb, sem)
        for q in range(NG):
            yield pltpu.make_async_copy(
                lw_hbm.at[wp.at[pl.ds(q * GCHUNK, GCHUNK)]],
                wb.at[pl.ds(q * GCHUNK, GCHUNK)], sem)

    def _issue(s, cp, wp, eb, wb, sem):
        _stage_idx(s, cp, wp)
        for c in _copies(cp, wp, eb, wb, sem):
            c.start()

    def _wait(cp, wp, eb, wb, sem):
        for c in _copies(cp, wp, eb, wb, sem):
            c.wait()

    def _compute(s, eb, wb):
        sb0 = pl.multiple_of(s * SB, SB)
        st0 = pl.multiple_of(s * TASKS, TASKS)

        @pl.loop(0, SB)
        def _per_b(b):
            ch = lax.shift_right_logical(
                plsc.load_gather(cidx, [zero16 + (sb0 + b)]), SH_HALF) & 1
            codd = ch == 1
            e = [jnp.where(codd,
                           eb[b, pl.ds(EMB + j * NLANE, NLANE)],
                           eb[b, pl.ds(j * NLANE, NLANE)])
                 for j in range(NV)]

            @pl.loop(0, KR)
            def _per_k(k):
                t = b * KR + k
                wh = lax.shift_right_logical(
                    plsc.load_gather(widx, [zero16 + (st0 + t)]), SH_HALF) & 1
                wodd = wh == 1
                p = jnp.where(wodd,
                              wb[t, pl.ds(EMB, NLANE)],
                              wb[t, pl.ds(0, NLANE)]) * e[0]
                for j in range(1, NV):
                    p = p + jnp.where(
                        wodd,
                        wb[t, pl.ds(EMB + j * NLANE, NLANE)],
                        wb[t, pl.ds(j * NLANE, NLANE)]) * e[j]
                pbuf[pl.ds(pl.multiple_of(t * NLANE, NLANE), NLANE)] = p

        @pl.loop(0, NGRP)
        def _per_g(g):
            base = g * (NLANE * NLANE) + lane * NLANE
            acc = plsc.load_gather(pbuf, [base])
            for j in range(1, NLANE):
                acc = acc + plsc.load_gather(pbuf, [base + j])
            dbuf[pl.ds(pl.multiple_of(g * NLANE, NLANE), NLANE)] = acc

        pltpu.sync_copy(dbuf, dots_hbm.at[pl.ds(t0 + st0, TASKS)])

    _issue(0, cp0, wp0, eb0, wb0, sem0)

    @pl.loop(0, NSUB // 2)
    def _pair(h):
        s0 = h * 2
        _issue(s0 + 1, cp1, wp1, eb1, wb1, sem1)
        _wait(cp0, wp0, eb0, wb0, sem0)
        _compute(s0, eb0, wb0)

        @pl.when(h < NSUB // 2 - 1)
        def _():
            _issue(s0 + 2, cp0, wp0, eb0, wb0, sem0)

        _wait(cp1, wp1, eb1, wb1, sem1)
        _compute(s0 + 1, eb1, wb1)


def _tc_loss_body(d_ref, o_ref):
    d = d_ref[...]
    col = lax.broadcasted_iota(jnp.int32, (B, KR), 1)
    act = jax.nn.sigmoid(d)
    pos = -jnp.log(act)
    neg = -jnp.log(1.0 - act + 1e-3)
    is_pos = col < K
    s_pos = jnp.sum(jnp.where(is_pos, pos, 0.0))
    s_neg = jnp.sum(jnp.where(is_pos, 0.0, neg))
    o_ref[0, 0] = s_pos / (B * K) + s_neg / (B * R)


_tc_loss = pl.pallas_call(
    _tc_loss_body,
    out_shape=jax.ShapeDtypeStruct((1, 1), jnp.float32),
    out_specs=pl.BlockSpec(memory_space=pltpu.SMEM),
)


def kernel(center, context, rand, embeddings, linear_w):
    center = center.astype(jnp.int32)
    cw = jnp.concatenate([context, rand], axis=1).astype(jnp.int32)
    embR = _tc_tpose(embeddings.T)
    lwR = _tc_tpose(linear_w.T)
    dots = _sc_dots(center, cw.reshape(-1), embR, lwR)
    loss = _tc_loss(dots.reshape(B, KR))
    return loss[0, 0]


# speedup vs baseline: 3.5946x; 1.1244x over previous
import functools

import jax
import jax.numpy as jnp
from jax import lax
from jax.experimental import pallas as pl
from jax.experimental.pallas import tpu as pltpu
from jax.experimental.pallas import tpu_sc as plsc

VOC = 1_000_000
EMB = 64
B = 16384
K = 20
R = 20
KR = K + R
PAIR = 2 * EMB

NC = 2
NS = 16
NW = NC * NS
NLANE = 16
NV = EMB // NLANE

NB = B // NW
SB = 8
NSUB = NB // SB
TASKS = SB * KR
GCHUNK = 64
NG = TASKS // GCHUNK
NGRP = TASKS // NLANE

TBLK = 32768
NTB = (VOC + TBLK - 1) // TBLK
HBLK = TBLK // 2
NPROW = NTB * HBLK
SH_BLK = TBLK.bit_length() - 1
SH_HALF = HBLK.bit_length() - 1


def _tc_tpose_body(x_ref, o_ref):
    x = x_ref[...]
    o_ref[...] = jnp.concatenate([x[:, :HBLK].T, x[:, HBLK:].T], axis=1)


_tc_tpose = pl.pallas_call(
    _tc_tpose_body,
    grid=(NTB,),
    in_specs=[pl.BlockSpec((EMB, TBLK), lambda i: (0, i))],
    out_specs=pl.BlockSpec((HBLK, PAIR), lambda i: (i, 0)),
    out_shape=jax.ShapeDtypeStruct((NPROW, PAIR), jnp.float32),
)


@functools.partial(
    pl.kernel,
    out_type=jax.ShapeDtypeStruct((B * KR,), jnp.float32),
    mesh=plsc.VectorSubcoreMesh(core_axis_name="c", subcore_axis_name="s"),
    compiler_params=pltpu.CompilerParams(
        needs_layout_passes=False, use_tc_tiling_on_sc=True),
    scratch_types=[
        pltpu.VMEM((NB,), jnp.int32),
        pltpu.VMEM((NB * KR,), jnp.int32),
        pltpu.VMEM((NLANE,), jnp.int32),
        pltpu.VMEM((NLANE,), jnp.int32),
        pltpu.VMEM((TASKS,), jnp.int32),
        pltpu.VMEM((TASKS,), jnp.int32),
        pltpu.VMEM((SB, PAIR), jnp.float32),
        pltpu.VMEM((SB, PAIR), jnp.float32),
        pltpu.VMEM((TASKS, PAIR), jnp.float32),
        pltpu.VMEM((TASKS, PAIR), jnp.float32),
        pltpu.VMEM((TASKS * NLANE,), jnp.float32),
        pltpu.VMEM((TASKS,), jnp.float32),
        pltpu.SemaphoreType.DMA,
        pltpu.SemaphoreType.DMA,
    ],
)
def _sc_dots(center_hbm, cw_hbm, emb_hbm, lw_hbm, dots_hbm,
             cidx, widx, cp0, cp1, wp0, wp1, eb0, eb1, wb0, wb1,
             pbuf, dbuf, sem0, sem1):
    wid = lax.axis_index("s") * NC + lax.axis_index("c")
    b0 = pl.multiple_of(wid * NB, NB)
    t0 = pl.multiple_of(wid * (NB * KR), NB * KR)
    pltpu.sync_copy(center_hbm.at[pl.ds(b0, NB)], cidx)
    pltpu.sync_copy(cw_hbm.at[pl.ds(t0, NB * KR)], widx)

    lane = lax.iota(jnp.int32, NLANE)
    zero16 = jnp.zeros((NLANE,), jnp.int32)

    def _pair_id(v):
        return (lax.shift_right_logical(v, SH_BLK) * HBLK) + (v & (HBLK - 1))

    def _stage_idx(s, cp, wp):
        sb0 = pl.multiple_of(s * SB, SB)
        st0 = pl.multiple_of(s * TASKS, TASKS)
        cp[:] = _pair_id(
            plsc.load_gather(cidx, [jnp.minimum(sb0 + lane, NB - 1)]))

        @pl.loop(0, NGRP)
        def _i(i):
            o = pl.multiple_of(i * NLANE, NLANE)
            wp[pl.ds(o, NLANE)] = _pair_id(widx[pl.ds(st0 + o, NLANE)])

    def _copies(cp, wp, eb, wb, sem):
        yield pltpu.make_async_copy(emb_hbm.at[cp.at[pl.ds(0, SB)]], eb, sem)
        for q in range(NG):
            yield pltpu.make_async_copy(
                lw_hbm.at[wp.at[pl.ds(q * GCHUNK, GCHUNK)]],
                wb.at[pl.ds(q * GCHUNK, GCHUNK)], sem)

    def _issue(s, cp, wp, eb, wb, sem):
        _stage_idx(s, cp, wp)
        for c in _copies(cp, wp, eb, wb, sem):
            c.start()

    def _wait(cp, wp, eb, wb, sem):
        for c in _copies(cp, wp, eb, wb, sem):
            c.wait()

    def _compute(s, eb, wb):
        sb0 = pl.multiple_of(s * SB, SB)
        st0 = pl.multiple_of(s * TASKS, TASKS)

        @pl.loop(0, SB)
        def _per_b(b):
            ch = lax.shift_right_logical(
                plsc.load_gather(cidx, [zero16 + (sb0 + b)]), SH_HALF) & 1
            codd = ch == 1
            e = [jnp.where(codd,
                           eb[b, pl.ds(EMB + j * NLANE, NLANE)],
                           eb[b, pl.ds(j * NLANE, NLANE)])
                 for j in range(NV)]

            @pl.loop(0, KR)
            def _per_k(k):
                t = b * KR + k
                wh = lax.shift_right_logical(
                    plsc.load_gather(widx, [zero16 + (st0 + t)]), SH_HALF) & 1
                wodd = wh == 1
                p = jnp.where(wodd,
                              wb[t, pl.ds(EMB, NLANE)],
                              wb[t, pl.ds(0, NLANE)]) * e[0]
                for j in range(1, NV):
                    p = p + jnp.where(
                        wodd,
                        wb[t, pl.ds(EMB + j * NLANE, NLANE)],
                        wb[t, pl.ds(j * NLANE, NLANE)]) * e[j]
                pbuf[pl.ds(pl.multiple_of(t * NLANE, NLANE), NLANE)] = p

        @pl.loop(0, NGRP)
        def _per_g(g):
            base = g * (NLANE * NLANE) + lane * NLANE
            acc = plsc.load_gather(pbuf, [base])
            for j in range(1, NLANE):
                acc = acc + plsc.load_gather(pbuf, [base + j])
            dbuf[pl.ds(pl.multiple_of(g * NLANE, NLANE), NLANE)] = acc

        pltpu.sync_copy(dbuf, dots_hbm.at[pl.ds(t0 + st0, TASKS)])

    _issue(0, cp0, wp0, eb0, wb0, sem0)

    @pl.loop(0, NSUB // 2)
    def _pair(h):
        s0 = h * 2
        _issue(s0 + 1, cp1, wp1, eb1, wb1, sem1)
        _wait(cp0, wp0, eb0, wb0, sem0)
        _compute(s0, eb0, wb0)

        @pl.when(h < NSUB // 2 - 1)
        def _():
            _issue(s0 + 2, cp0, wp0, eb0, wb0, sem0)

        _wait(cp1, wp1, eb1, wb1, sem1)
        _compute(s0 + 1, eb1, wb1)


def _tc_loss_body(d_ref, o_ref):
    d = d_ref[...]
    col = lax.broadcasted_iota(jnp.int32, (B, KR), 1)
    act = jax.nn.sigmoid(d)
    pos = -jnp.log(act)
    neg = -jnp.log(1.0 - act + 1e-3)
    is_pos = col < K
    s_pos = jnp.sum(jnp.where(is_pos, pos, 0.0))
    s_neg = jnp.sum(jnp.where(is_pos, 0.0, neg))
    o_ref[0, 0] = s_pos / (B * K) + s_neg / (B * R)


_tc_loss = pl.pallas_call(
    _tc_loss_body,
    out_shape=jax.ShapeDtypeStruct((1, 1), jnp.float32),
    out_specs=pl.BlockSpec(memory_space=pltpu.SMEM),
)


def kernel(center, context, rand, embeddings, linear_w):
    center = center.astype(jnp.int32)
    cw = jnp.concatenate([context, rand], axis=1).astype(jnp.int32)
    embR = _tc_tpose(embeddings.T)
    lwR = _tc_tpose(linear_w.T)
    dots = _sc_dots(center, cw.reshape(-1), embR, lwR)
    loss = _tc_loss(dots.reshape(B, KR))
    return loss[0, 0]


# speedup vs baseline: 3.6009x; 1.0017x over previous
import functools

import jax
import jax.numpy as jnp
from jax import lax
from jax.experimental import pallas as pl
from jax.experimental.pallas import tpu as pltpu
from jax.experimental.pallas import tpu_sc as plsc

VOC = 1_000_000
EMB = 64
B = 16384
K = 20
R = 20
KR = K + R
PAIR = 2 * EMB

NC = 2
NS = 16
NW = NC * NS
NLANE = 16
NV = EMB // NLANE

NB = B // NW
SB = 8
NSUB = NB // SB
TASKS = SB * KR
GCHUNKS = (128, 128, 64)
NGRP = TASKS // NLANE

TBLK = 32768
NTB = (VOC + TBLK - 1) // TBLK
HBLK = TBLK // 2
NPROW = NTB * HBLK
SH_BLK = TBLK.bit_length() - 1
SH_HALF = HBLK.bit_length() - 1


def _tc_tpose_body(x_ref, o_ref):
    x = x_ref[...]
    o_ref[...] = jnp.concatenate([x[:, :HBLK].T, x[:, HBLK:].T], axis=1)


_tc_tpose = pl.pallas_call(
    _tc_tpose_body,
    grid=(NTB,),
    in_specs=[pl.BlockSpec((EMB, TBLK), lambda i: (0, i))],
    out_specs=pl.BlockSpec((HBLK, PAIR), lambda i: (i, 0)),
    out_shape=jax.ShapeDtypeStruct((NPROW, PAIR), jnp.float32),
)


@functools.partial(
    pl.kernel,
    out_type=jax.ShapeDtypeStruct((B * KR,), jnp.float32),
    mesh=plsc.VectorSubcoreMesh(core_axis_name="c", subcore_axis_name="s"),
    compiler_params=pltpu.CompilerParams(
        needs_layout_passes=False, use_tc_tiling_on_sc=True),
    scratch_types=[
        pltpu.VMEM((NB,), jnp.int32),
        pltpu.VMEM((NB * KR,), jnp.int32),
        pltpu.VMEM((NLANE,), jnp.int32),
        pltpu.VMEM((NLANE,), jnp.int32),
        pltpu.VMEM((TASKS,), jnp.int32),
        pltpu.VMEM((TASKS,), jnp.int32),
        pltpu.VMEM((SB, PAIR), jnp.float32),
        pltpu.VMEM((SB, PAIR), jnp.float32),
        pltpu.VMEM((TASKS, PAIR), jnp.float32),
        pltpu.VMEM((TASKS, PAIR), jnp.float32),
        pltpu.VMEM((TASKS * NLANE,), jnp.float32),
        pltpu.VMEM((TASKS,), jnp.float32),
        pltpu.SemaphoreType.DMA,
        pltpu.SemaphoreType.DMA,
    ],
)
def _sc_dots(center_hbm, cw_hbm, emb_hbm, lw_hbm, dots_hbm,
             cidx, widx, cp0, cp1, wp0, wp1, eb0, eb1, wb0, wb1,
             pbuf, dbuf, sem0, sem1):
    wid = lax.axis_index("s") * NC + lax.axis_index("c")
    b0 = pl.multiple_of(wid * NB, NB)
    t0 = pl.multiple_of(wid * (NB * KR), NB * KR)
    pltpu.sync_copy(center_hbm.at[pl.ds(b0, NB)], cidx)
    pltpu.sync_copy(cw_hbm.at[pl.ds(t0, NB * KR)], widx)

    lane = lax.iota(jnp.int32, NLANE)
    zero16 = jnp.zeros((NLANE,), jnp.int32)

    def _pair_id(v):
        return (lax.shift_right_logical(v, SH_BLK) * HBLK) + (v & (HBLK - 1))

    def _stage_idx(s, cp, wp):
        sb0 = pl.multiple_of(s * SB, SB)
        st0 = pl.multiple_of(s * TASKS, TASKS)
        cp[:] = _pair_id(
            plsc.load_gather(cidx, [jnp.minimum(sb0 + lane, NB - 1)]))

        @pl.loop(0, NGRP)
        def _i(i):
            o = pl.multiple_of(i * NLANE, NLANE)
            wp[pl.ds(o, NLANE)] = _pair_id(widx[pl.ds(st0 + o, NLANE)])

    def _copies(cp, wp, eb, wb, sem):
        yield pltpu.make_async_copy(emb_hbm.at[cp.at[pl.ds(0, SB)]], eb, sem)
        o = 0
        for g in GCHUNKS:
            yield pltpu.make_async_copy(
                lw_hbm.at[wp.at[pl.ds(o, g)]], wb.at[pl.ds(o, g)], sem)
            o += g

    def _issue(s, cp, wp, eb, wb, sem):
        _stage_idx(s, cp, wp)
        for c in _copies(cp, wp, eb, wb, sem):
            c.start()

    def _wait(cp, wp, eb, wb, sem):
        for c in _copies(cp, wp, eb, wb, sem):
            c.wait()

    def _compute(s, eb, wb):
        sb0 = pl.multiple_of(s * SB, SB)
        st0 = pl.multiple_of(s * TASKS, TASKS)

        @pl.loop(0, SB)
        def _per_b(b):
            ch = lax.shift_right_logical(
                plsc.load_gather(cidx, [zero16 + (sb0 + b)]), SH_HALF) & 1
            codd = ch == 1
            e = [jnp.where(codd,
                           eb[b, pl.ds(EMB + j * NLANE, NLANE)],
                           eb[b, pl.ds(j * NLANE, NLANE)])
                 for j in range(NV)]

            @pl.loop(0, KR)
            def _per_k(k):
                t = b * KR + k
                wh = lax.shift_right_logical(
                    plsc.load_gather(widx, [zero16 + (st0 + t)]), SH_HALF) & 1
                wodd = wh == 1
                p = jnp.where(wodd,
                              wb[t, pl.ds(EMB, NLANE)],
                              wb[t, pl.ds(0, NLANE)]) * e[0]
                for j in range(1, NV):
                    p = p + jnp.where(
                        wodd,
                        wb[t, pl.ds(EMB + j * NLANE, NLANE)],
                        wb[t, pl.ds(j * NLANE, NLANE)]) * e[j]
                pbuf[pl.ds(pl.multiple_of(t * NLANE, NLANE), NLANE)] = p

        @pl.loop(0, NGRP)
        def _per_g(g):
            base = g * (NLANE * NLANE) + lane * NLANE
            acc = plsc.load_gather(pbuf, [base])
            for j in range(1, NLANE):
                acc = acc + plsc.load_gather(pbuf, [base + j])
            dbuf[pl.ds(pl.multiple_of(g * NLANE, NLANE), NLANE)] = acc

        pltpu.sync_copy(dbuf, dots_hbm.at[pl.ds(t0 + st0, TASKS)])

    _issue(0, cp0, wp0, eb0, wb0, sem0)

    @pl.loop(0, NSUB // 2)
    def _pair(h):
        s0 = h * 2
        _issue(s0 + 1, cp1, wp1, eb1, wb1, sem1)
        _wait(cp0, wp0, eb0, wb0, sem0)
        _compute(s0, eb0, wb0)

        @pl.when(h < NSUB // 2 - 1)
        def _():
            _issue(s0 + 2, cp0, wp0, eb0, wb0, sem0)

        _wait(cp1, wp1, eb1, wb1, sem1)
        _compute(s0 + 1, eb1, wb1)


def _tc_loss_body(d_ref, o_ref):
    d = d_ref[...]
    col = lax.broadcasted_iota(jnp.int32, (B, KR), 1)
    act = jax.nn.sigmoid(d)
    pos = -jnp.log(act)
    neg = -jnp.log(1.0 - act + 1e-3)
    is_pos = col < K
    s_pos = jnp.sum(jnp.where(is_pos, pos, 0.0))
    s_neg = jnp.sum(jnp.where(is_pos, 0.0, neg))
    o_ref[0, 0] = s_pos / (B * K) + s_neg / (B * R)


_tc_loss = pl.pallas_call(
    _tc_loss_body,
    out_shape=jax.ShapeDtypeStruct((1, 1), jnp.float32),
    out_specs=pl.BlockSpec(memory_space=pltpu.SMEM),
)


def kernel(center, context, rand, embeddings, linear_w):
    center = center.astype(jnp.int32)
    cw = jnp.concatenate([context, rand], axis=1).astype(jnp.int32)
    embR = _tc_tpose(embeddings.T)
    lwR = _tc_tpose(linear_w.T)
    dots = _sc_dots(center, cw.reshape(-1), embR, lwR)
    loss = _tc_loss(dots.reshape(B, KR))
    return loss[0, 0]
